# Initial kernel scaffold; baseline (speedup 1.0000x reference)
import jax
import jax.numpy as jnp
from jax.experimental import pallas as pl


def kernel(x_real, x_imag, gamma_r, gamma_i, beta_r, beta_i):
    raise NotImplementedError("write your pallas kernel here")


# speedup vs baseline: 5.0373x; 5.0373x over previous
import jax
import jax.numpy as jnp
from jax.experimental import pallas as pl
from jax.experimental.pallas import tpu as pltpu

_EPS = 1e-4


def _stats_kernel(xr_ref, xi_ref, stats_ref):
    j = pl.program_id(1)
    xr = xr_ref[...]
    xi = xi_ref[...]
    tr = jnp.sum(xr, axis=0)
    ti = jnp.sum(xi, axis=0)
    srr = jnp.sum(xr * xr, axis=(0, 1))
    sii = jnp.sum(xi * xi, axis=(0, 1))
    sri = jnp.sum(xr * xi, axis=(0, 1))
    urr = jnp.sum(tr * tr, axis=0)
    uii = jnp.sum(ti * ti, axis=0)
    uri = jnp.sum(tr * ti, axis=0)
    z = jnp.zeros_like(srr)
    upd = jnp.stack([srr, sii, sri, urr, uii, uri, z, z], axis=0)[None]

    @pl.when(j == 0)
    def _():
        stats_ref[...] = upd

    @pl.when(j != 0)
    def _():
        stats_ref[...] += upd


def _make_apply_kernel(n_total, inv_b):
    inv_nm1 = 1.0 / (n_total - 1)

    def _apply_kernel(xr_ref, xi_ref, stats_ref, gr_ref, gi_ref, br_ref,
                      bi_ref, or_ref, oi_ref):
        stats = stats_ref[0] + stats_ref[1]
        srr, sii, sri = stats[0], stats[1], stats[2]
        urr, uii, uri = stats[3], stats[4], stats[5]
        a = (srr - urr * inv_b) * inv_nm1 + _EPS
        c = (sii - uii * inv_b) * inv_nm1 + _EPS
        b = (sri - uri * inv_b) * inv_nm1
        det = a * c - b * b
        s = jnp.sqrt(det)
        k = jax.lax.rsqrt(det * (a + c + 2.0 * s))
        w_rr = (c + s) * k
        w_ii = (a + s) * k
        w_ri = -b * k
        gr = gr_ref[0]
        gi = gi_ref[0]
        crr = gr * w_rr - gi * w_ri
        cri = gr * w_ri - gi * w_ii
        cir = gr * w_ri + gi * w_rr
        cii = gr * w_ii + gi * w_ri

        xr = xr_ref[...]
        xi = xi_ref[...]
        f = xr.shape[-1]
        mr = jnp.sum(xr, axis=2, keepdims=True) * (1.0 / f)
        mi = jnp.sum(xi, axis=2, keepdims=True) * (1.0 / f)
        xrc = xr - mr
        xic = xi - mi
        or_ref[...] = crr * xrc + cri * xic + br_ref[0]
        oi_ref[...] = cir * xrc + cii * xic + bi_ref[0]

    return _apply_kernel


def kernel(x_real, x_imag, gamma_r, gamma_i, beta_r, beta_i):
    B, C, F = x_real.shape
    CC = 8
    NCORE = 2
    nc = C // CC
    half = nc // NCORE

    x_spec = pl.BlockSpec((B, CC, F), lambda i, j: (0, i * half + j, 0))
    stats = pl.pallas_call(
        _stats_kernel,
        grid=(NCORE, half),
        in_specs=[x_spec, x_spec],
        out_specs=pl.BlockSpec((1, 8, F), lambda i, j: (i, 0, 0)),
        out_shape=jax.ShapeDtypeStruct((NCORE, 8, F), jnp.float32),
        compiler_params=pltpu.CompilerParams(
            dimension_semantics=("parallel", "arbitrary"),
            vmem_limit_bytes=48 * 1024 * 1024,
        ),
        name="cln_stats",
    )(x_real, x_imag)

    BB = 4
    xb_spec = pl.BlockSpec((BB, C, F), lambda i: (i, 0, 0))
    vec_spec = pl.BlockSpec((1, F), lambda i: (0, 0))
    out_r, out_i = pl.pallas_call(
        _make_apply_kernel(B * C, 1.0 / B),
        grid=(B // BB,),
        in_specs=[
            xb_spec,
            xb_spec,
            pl.BlockSpec((NCORE, 8, F), lambda i: (0, 0, 0)),
            vec_spec, vec_spec, vec_spec, vec_spec,
        ],
        out_specs=[xb_spec, xb_spec],
        out_shape=[jax.ShapeDtypeStruct((B, C, F), jnp.float32)] * 2,
        compiler_params=pltpu.CompilerParams(
            dimension_semantics=("parallel",),
            vmem_limit_bytes=48 * 1024 * 1024,
        ),
        name="cln_apply",
    )(x_real, x_imag, stats,
      gamma_r.reshape(1, F), gamma_i.reshape(1, F),
      beta_r.reshape(1, F), beta_i.reshape(1, F))

    return jnp.stack([out_r, out_i], axis=-1)
